# baseline (speedup 1.0000x reference)
import jax
import jax.numpy as jnp
from jax import lax
from jax.experimental import pallas as pl
from jax.experimental.pallas import tpu as pltpu

K = 16


def _fold_max(x):
    while x.shape[0] > 8:
        h = x.shape[0] // 2
        x = jnp.maximum(x[:h], x[h:])
    return jnp.max(x, axis=0)


def _fold_min(x):
    while x.shape[0] > 8:
        h = x.shape[0] // 2
        x = jnp.minimum(x[:h], x[h:])
    return jnp.min(x, axis=0)


def _topk_kernel(act_ref, sens_ref, vals_ref, idxs_ref):
    act = act_ref[0]
    sens = sens_ref[0]
    A, L = act.shape
    S = sens.shape[0]

    xn = jnp.sqrt(jnp.sum(act * act, axis=1))
    yn = jnp.sqrt(jnp.sum(sens * sens, axis=1))

    num_t = lax.dot_general(
        act, sens,
        dimension_numbers=(((1,), (1,)), ((), ())),
        precision=lax.Precision.DEFAULT,
        preferred_element_type=jnp.float32,
    ).T
    sim = num_t / (yn[:, None] * xn[None, :])

    score = sim * sim
    ikey = lax.shift_left(lax.bitcast_convert_type(score, jnp.int32), 1)
    ikey = ikey | lax.shift_right_logical(
        lax.bitcast_convert_type(sim, jnp.int32), 31)
    iota = lax.broadcasted_iota(jnp.int32, (S, A), 0)
    big = jnp.int32(1 << 30)
    for j in range(K):
        m = _fold_max(ikey)
        cand = jnp.where(ikey == m[None, :], iota, big)
        idx = _fold_min(cand)
        r = jnp.sqrt(lax.bitcast_convert_type(
            lax.shift_right_logical(m, 1), jnp.float32))
        val = jnp.where((m & 1) == 1, -r, r)
        vals_ref[0, j, :] = val
        idxs_ref[0, j, :] = idx
        ikey = jnp.where(iota == idx[None, :], -1, ikey)


@jax.jit
def kernel(x_actuators, x_sensors):
    B, A, L = x_actuators.shape
    S = x_sensors.shape[1]
    k = K

    vals_t, idxs_t = pl.pallas_call(
        _topk_kernel,
        grid=(B,),
        compiler_params=pltpu.CompilerParams(
            dimension_semantics=("parallel",),
        ),
        in_specs=[
            pl.BlockSpec((1, A, L), lambda b: (b, 0, 0)),
            pl.BlockSpec((1, S, L), lambda b: (b, 0, 0)),
        ],
        out_specs=[
            pl.BlockSpec((1, k, A), lambda b: (b, 0, 0)),
            pl.BlockSpec((1, k, A), lambda b: (b, 0, 0)),
        ],
        out_shape=[
            jax.ShapeDtypeStruct((B, k, A), jnp.float32),
            jax.ShapeDtypeStruct((B, k, A), jnp.int32),
        ],
    )(x_actuators, x_sensors)

    target_nodes = jnp.swapaxes(idxs_t, 1, 2).reshape(B, A * k)
    source_nodes = jnp.tile(jnp.repeat(jnp.arange(A), k)[None, :], (B, 1))
    edges = jnp.stack([source_nodes, target_nodes], axis=1)
    weights = jnp.swapaxes(vals_t, 1, 2).reshape(B, A * k)
    return edges, weights


# speedup vs baseline: 1.1229x; 1.1229x over previous
import jax
import jax.numpy as jnp
from jax import lax
from jax.experimental import pallas as pl
from jax.experimental.pallas import tpu as pltpu

K = 16


def _topk_kernel(act_ref, sens_ref, vals_ref, idxs_ref):
    act = act_ref[0]
    sens = sens_ref[0]
    A, L = act.shape
    S = sens.shape[0]

    xn = jnp.sqrt(jnp.sum(act * act, axis=1))
    yn = jnp.sqrt(jnp.sum(sens * sens, axis=1))

    num_t = lax.dot_general(
        act, sens,
        dimension_numbers=(((1,), (1,)), ((), ())),
        precision=lax.Precision.DEFAULT,
        preferred_element_type=jnp.float32,
    ).T
    sim = num_t / (yn[:, None] * xn[None, :])

    score = sim * sim
    iota = lax.broadcasted_iota(jnp.int32, (S, A), 0)
    sign = lax.shift_right_logical(
        lax.bitcast_convert_type(sim, jnp.int32), 31)
    fpacked = (iota * 2 + sign).astype(jnp.float32)
    bigf = jnp.float32(1e9)
    for j in range(K):
        m = jnp.max(score, axis=0)
        cand = jnp.where(score >= m[None, :], fpacked, bigf)
        idxp_f = jnp.min(cand, axis=0)
        idxp = idxp_f.astype(jnp.int32)
        r = jnp.sqrt(m)
        val = jnp.where((idxp & 1) == 1, -r, r)
        vals_ref[0, j, :] = val
        idxs_ref[0, j, :] = lax.shift_right_logical(idxp, 1)
        score = jnp.where(fpacked == idxp_f[None, :], -1.0, score)


@jax.jit
def kernel(x_actuators, x_sensors):
    B, A, L = x_actuators.shape
    S = x_sensors.shape[1]
    k = K

    vals_t, idxs_t = pl.pallas_call(
        _topk_kernel,
        grid=(B,),
        compiler_params=pltpu.CompilerParams(
            dimension_semantics=("parallel",),
        ),
        in_specs=[
            pl.BlockSpec((1, A, L), lambda b: (b, 0, 0)),
            pl.BlockSpec((1, S, L), lambda b: (b, 0, 0)),
        ],
        out_specs=[
            pl.BlockSpec((1, k, A), lambda b: (b, 0, 0)),
            pl.BlockSpec((1, k, A), lambda b: (b, 0, 0)),
        ],
        out_shape=[
            jax.ShapeDtypeStruct((B, k, A), jnp.float32),
            jax.ShapeDtypeStruct((B, k, A), jnp.int32),
        ],
    )(x_actuators, x_sensors)

    target_nodes = jnp.swapaxes(idxs_t, 1, 2).reshape(B, A * k)
    source_nodes = jnp.tile(jnp.repeat(jnp.arange(A), k)[None, :], (B, 1))
    edges = jnp.stack([source_nodes, target_nodes], axis=1)
    weights = jnp.swapaxes(vals_t, 1, 2).reshape(B, A * k)
    return edges, weights


# speedup vs baseline: 1.3034x; 1.1607x over previous
import jax
import jax.numpy as jnp
from jax import lax
from jax.experimental import pallas as pl
from jax.experimental.pallas import tpu as pltpu

K = 16


def _topk_kernel(act_ref, sens_ref, vals_ref, idxs_ref):
    act = act_ref[0]
    sens = sens_ref[0]
    A, L = act.shape
    S = sens.shape[0]

    xn = jnp.sqrt(jnp.sum(act * act, axis=1))
    yn = jnp.sqrt(jnp.sum(sens * sens, axis=1))

    num_t = lax.dot_general(
        act, sens,
        dimension_numbers=(((1,), (1,)), ((), ())),
        precision=lax.Precision.DEFAULT,
        preferred_element_type=jnp.float32,
    ).T
    sim = num_t / (yn[:, None] * xn[None, :])

    score = sim * sim
    iota = lax.broadcasted_iota(jnp.int32, (S, A), 0)
    sign = lax.shift_right_logical(
        lax.bitcast_convert_type(sim, jnp.int32), 31)
    fpacked = (iota * 2 + sign).astype(jnp.float32)
    bigf = jnp.float32(1e9)

    R = 32
    NS = S // R
    sc_slabs = [score[r * R:(r + 1) * R] for r in range(NS)]
    fp_slabs = [fpacked[r * R:(r + 1) * R] for r in range(NS)]

    idxp_f = None
    for j in range(K):
        acc = None
        for r in range(NS):
            s = sc_slabs[r]
            if idxp_f is not None:
                s = jnp.where(fp_slabs[r] == idxp_f[None, :], -1.0, s)
                sc_slabs[r] = s
            acc = s if acc is None else jnp.maximum(acc, s)
        m = jnp.max(acc, axis=0)

        acc2 = None
        for r in range(NS):
            c = jnp.where(sc_slabs[r] >= m[None, :], fp_slabs[r], bigf)
            acc2 = c if acc2 is None else jnp.minimum(acc2, c)
        idxp_f = jnp.min(acc2, axis=0)

        idxp = idxp_f.astype(jnp.int32)
        rt = jnp.sqrt(m)
        val = jnp.where((idxp & 1) == 1, -rt, rt)
        vals_ref[0, j, :] = val
        idxs_ref[0, j, :] = lax.shift_right_logical(idxp, 1)


@jax.jit
def kernel(x_actuators, x_sensors):
    B, A, L = x_actuators.shape
    S = x_sensors.shape[1]
    k = K

    vals_t, idxs_t = pl.pallas_call(
        _topk_kernel,
        grid=(B,),
        compiler_params=pltpu.CompilerParams(
            dimension_semantics=("parallel",),
        ),
        in_specs=[
            pl.BlockSpec((1, A, L), lambda b: (b, 0, 0)),
            pl.BlockSpec((1, S, L), lambda b: (b, 0, 0)),
        ],
        out_specs=[
            pl.BlockSpec((1, k, A), lambda b: (b, 0, 0)),
            pl.BlockSpec((1, k, A), lambda b: (b, 0, 0)),
        ],
        out_shape=[
            jax.ShapeDtypeStruct((B, k, A), jnp.float32),
            jax.ShapeDtypeStruct((B, k, A), jnp.int32),
        ],
    )(x_actuators, x_sensors)

    target_nodes = jnp.swapaxes(idxs_t, 1, 2).reshape(B, A * k)
    source_nodes = jnp.tile(jnp.repeat(jnp.arange(A), k)[None, :], (B, 1))
    edges = jnp.stack([source_nodes, target_nodes], axis=1)
    weights = jnp.swapaxes(vals_t, 1, 2).reshape(B, A * k)
    return edges, weights


# speedup vs baseline: 1.3433x; 1.0306x over previous
import jax
import jax.numpy as jnp
from jax import lax
from jax.experimental import pallas as pl
from jax.experimental.pallas import tpu as pltpu

K = 16


def _topk_kernel(act_ref, sens_ref, vals_ref, idxs_ref):
    act = act_ref[0]
    sens = sens_ref[0]
    A, L = act.shape
    S = sens.shape[0]

    xn = jnp.sqrt(jnp.sum(act * act, axis=1))
    yn = jnp.sqrt(jnp.sum(sens * sens, axis=1))

    num_t = lax.dot_general(
        sens, act,
        dimension_numbers=(((1,), (1,)), ((), ())),
        precision=lax.Precision.DEFAULT,
        preferred_element_type=jnp.float32,
    )
    sim = num_t / (yn[:, None] * xn[None, :])

    score = sim * sim
    iota = lax.broadcasted_iota(jnp.int32, (S, A), 0)
    sign = lax.shift_right_logical(
        lax.bitcast_convert_type(sim, jnp.int32), 31)
    fpacked = (iota * 2 + sign).astype(jnp.float32)
    bigf = jnp.float32(1e9)

    R = 32
    NS = S // R
    sc_slabs = [score[r * R:(r + 1) * R] for r in range(NS)]
    fp_slabs = [fpacked[r * R:(r + 1) * R] for r in range(NS)]

    idxp_f = None
    for j in range(K):
        acc = None
        for r in range(NS):
            s = sc_slabs[r]
            if idxp_f is not None:
                s = jnp.where(fp_slabs[r] == idxp_f[None, :], -1.0, s)
                sc_slabs[r] = s
            acc = s if acc is None else jnp.maximum(acc, s)
        m = jnp.max(acc, axis=0)

        acc2 = None
        for r in range(NS):
            c = jnp.where(sc_slabs[r] >= m[None, :], fp_slabs[r], bigf)
            acc2 = c if acc2 is None else jnp.minimum(acc2, c)
        idxp_f = jnp.min(acc2, axis=0)

        idxp = idxp_f.astype(jnp.int32)
        rt = jnp.sqrt(m)
        val = jnp.where((idxp & 1) == 1, -rt, rt)
        vals_ref[0, j, :] = val
        idxs_ref[0, j, :] = lax.shift_right_logical(idxp, 1)


@jax.jit
def kernel(x_actuators, x_sensors):
    B, A, L = x_actuators.shape
    S = x_sensors.shape[1]
    k = K

    vals_t, idxs_t = pl.pallas_call(
        _topk_kernel,
        grid=(B,),
        compiler_params=pltpu.CompilerParams(
            dimension_semantics=("arbitrary",),
        ),
        in_specs=[
            pl.BlockSpec((1, A, L), lambda b: (b, 0, 0)),
            pl.BlockSpec((1, S, L), lambda b: (b, 0, 0)),
        ],
        out_specs=[
            pl.BlockSpec((1, k, A), lambda b: (b, 0, 0)),
            pl.BlockSpec((1, k, A), lambda b: (b, 0, 0)),
        ],
        out_shape=[
            jax.ShapeDtypeStruct((B, k, A), jnp.float32),
            jax.ShapeDtypeStruct((B, k, A), jnp.int32),
        ],
    )(x_actuators, x_sensors)

    target_nodes = jnp.swapaxes(idxs_t, 1, 2).reshape(B, A * k)
    source_nodes = jnp.tile(jnp.repeat(jnp.arange(A), k)[None, :], (B, 1))
    edges = jnp.stack([source_nodes, target_nodes], axis=1)
    weights = jnp.swapaxes(vals_t, 1, 2).reshape(B, A * k)
    return edges, weights
